# Initial kernel scaffold; baseline (speedup 1.0000x reference)
import jax
import jax.numpy as jnp
from jax.experimental import pallas as pl


def kernel(x, inf_embed):
    raise NotImplementedError("write your pallas kernel here")


# speedup vs baseline: 1.1845x; 1.1845x over previous
import functools

import jax
import jax.numpy as jnp
from jax import lax
from jax.experimental import pallas as pl
from jax.experimental.pallas import tpu as pltpu
from jax.experimental.pallas import tpu_sc as plsc

NW = 32
ROWS_PER_DMA = 128


@functools.lru_cache(maxsize=None)
def _build(n_pad: int, steps: int, v: int, d: int):
    mesh = plsc.VectorSubcoreMesh(core_axis_name="c", subcore_axis_name="s")
    b_per_w = steps * ROWS_PER_DMA

    @functools.partial(
        pl.kernel,
        mesh=mesh,
        out_type=jax.ShapeDtypeStruct((n_pad, d), jnp.float32),
        scratch_types=[
            pltpu.VMEM((b_per_w,), jnp.int32),
            pltpu.VMEM((ROWS_PER_DMA, d), jnp.float32),
            pltpu.VMEM((ROWS_PER_DMA, d), jnp.float32),
            pltpu.SemaphoreType.DMA,
            pltpu.SemaphoreType.DMA,
        ],
    )
    def gather_kernel(idx_hbm, table_hbm, out_hbm, idx_v, rows0, rows1, sem0, sem1):
        cid = lax.axis_index("c")
        sid = lax.axis_index("s")
        wid = sid * 2 + cid
        base = wid * b_per_w
        pltpu.sync_copy(idx_hbm.at[pl.ds(base, b_per_w)], idx_v)

        rows = (rows0, rows1)
        sems = (sem0, sem1)
        pltpu.async_copy(table_hbm.at[idx_v.at[pl.ds(0, ROWS_PER_DMA)]], rows0, sem0)

        def body(s, _):
            def step(c):
                @pl.when(s + 1 < steps)
                def _():
                    pltpu.async_copy(
                        table_hbm.at[
                            idx_v.at[pl.ds((s + 1) * ROWS_PER_DMA, ROWS_PER_DMA)]
                        ],
                        rows[1 - c],
                        sems[1 - c],
                    )
                pltpu.make_async_copy(
                    table_hbm.at[idx_v.at[pl.ds(s * ROWS_PER_DMA, ROWS_PER_DMA)]],
                    rows[c],
                    sems[c],
                ).wait()
                pltpu.sync_copy(
                    rows[c],
                    out_hbm.at[pl.ds(base + s * ROWS_PER_DMA, ROWS_PER_DMA)],
                )

            cur = lax.rem(s, 2)

            @pl.when(cur == 0)
            def _():
                step(0)

            @pl.when(cur == 1)
            def _():
                step(1)

            return _

        lax.fori_loop(0, steps, body, None)

    return gather_kernel


def kernel(x, inf_embed):
    n = x.shape[0]
    v, d = inf_embed.shape
    chunk = NW * ROWS_PER_DMA
    steps = -(-n // chunk)
    n_pad = steps * chunk
    idx = x.astype(jnp.int32)
    if n_pad != n:
        idx = jnp.concatenate([idx, jnp.zeros((n_pad - n,), jnp.int32)])
    out = _build(n_pad, steps, v, d)(idx, inf_embed)
    return out[:n]


# speedup vs baseline: 3.6350x; 3.0689x over previous
import functools

import jax
import jax.numpy as jnp
from jax import lax
from jax.experimental import pallas as pl
from jax.experimental.pallas import tpu as pltpu
from jax.experimental.pallas import tpu_sc as plsc

NW = 32
RB = 128


@functools.lru_cache(maxsize=None)
def _build(n: int, v: int, d: int):
    assert n % 8 == 0 and n >= NW * RB
    nblk, rem = divmod(n, RB)
    bs, extra = divmod(nblk, NW)
    stage = (bs + 1) * RB
    off_extra = extra * stage
    o_last = off_extra + (NW - 1 - extra) * bs * RB if extra < NW else (NW - 1) * stage
    tail_off = o_last + bs * RB
    pad_to = o_last + stage
    par = bs % 2

    mesh = plsc.VectorSubcoreMesh(core_axis_name="c", subcore_axis_name="s")

    @functools.partial(
        pl.kernel,
        mesh=mesh,
        out_type=jax.ShapeDtypeStruct((n, d), jnp.float32),
        scratch_types=[
            pltpu.VMEM((stage,), jnp.int32),
            pltpu.VMEM((RB, d), jnp.float32),
            pltpu.VMEM((RB, d), jnp.float32),
            pltpu.SemaphoreType.DMA,
            pltpu.SemaphoreType.DMA,
        ],
    )
    def gather_kernel(idx_hbm, table_hbm, out_hbm, idx_v, rows0, rows1, sem0, sem1):
        cid = lax.axis_index("c")
        sid = lax.axis_index("s")
        wid = sid * 2 + cid
        is_extra = wid < extra
        nfull = jnp.where(is_extra, bs + 1, bs)
        base = jnp.where(
            is_extra, wid * stage, off_extra + (wid - extra) * (bs * RB)
        )
        base = pl.multiple_of(base, 8)
        pltpu.sync_copy(idx_hbm.at[pl.ds(base, stage)], idx_v)

        rows = (rows0, rows1)
        sems = (sem0, sem1)
        pltpu.async_copy(table_hbm.at[idx_v.at[pl.ds(0, RB)]], rows0, sem0)

        def body(s, _):
            def step(c):
                @pl.when(s + 1 < nfull)
                def _():
                    pltpu.async_copy(
                        table_hbm.at[idx_v.at[pl.ds((s + 1) * RB, RB)]],
                        rows[1 - c],
                        sems[1 - c],
                    )
                pltpu.make_async_copy(
                    table_hbm.at[idx_v.at[pl.ds(s * RB, RB)]],
                    rows[c],
                    sems[c],
                ).wait()
                pltpu.sync_copy(
                    rows[c], out_hbm.at[pl.ds(base + s * RB, RB)]
                )

            cur = lax.rem(s, 2)

            @pl.when(cur == 0)
            def _():
                step(0)

            @pl.when(cur == 1)
            def _():
                step(1)

            return _

        lax.fori_loop(0, nfull, body, None)

        if rem:
            @pl.when(wid == NW - 1)
            def _():
                pltpu.async_copy(
                    table_hbm.at[idx_v.at[pl.ds(bs * RB, rem)]],
                    rows[par].at[pl.ds(0, rem)],
                    sems[par],
                )
                pltpu.make_async_copy(
                    table_hbm.at[idx_v.at[pl.ds(bs * RB, rem)]],
                    rows[par].at[pl.ds(0, rem)],
                    sems[par],
                ).wait()
                pltpu.sync_copy(
                    rows[par].at[pl.ds(0, rem)],
                    out_hbm.at[pl.ds(tail_off, rem)],
                )

    return gather_kernel, pad_to


def kernel(x, inf_embed):
    n = x.shape[0]
    v, d = inf_embed.shape
    idx = x.astype(jnp.int32)
    gk, pad_to = _build(n, v, d)
    if pad_to > n:
        idx = jnp.concatenate([idx, jnp.zeros((pad_to - n,), jnp.int32)])
    return gk(idx, inf_embed)


# speedup vs baseline: 5.5470x; 1.5260x over previous
import functools

import jax
import jax.numpy as jnp
from jax import lax
from jax.experimental import pallas as pl
from jax.experimental.pallas import tpu as pltpu
from jax.experimental.pallas import tpu_sc as plsc

NW = 32
RB = 128


@functools.lru_cache(maxsize=None)
def _build(n: int, v: int, d: int):
    assert n % 8 == 0 and n >= NW * RB
    nblk, rem = divmod(n, RB)
    bs, extra = divmod(nblk, NW)
    stage = (bs + 1) * RB
    off_extra = extra * stage
    o_last = off_extra + (NW - 1 - extra) * bs * RB if extra < NW else (NW - 1) * stage
    tail_off = o_last + bs * RB
    pad_to = o_last + stage
    par = bs % 2

    mesh = plsc.VectorSubcoreMesh(core_axis_name="c", subcore_axis_name="s")

    @functools.partial(
        pl.kernel,
        mesh=mesh,
        out_type=jax.ShapeDtypeStruct((n, d), jnp.float32),
        scratch_types=[
            pltpu.VMEM((stage,), jnp.int32),
            pltpu.VMEM((RB, d), jnp.float32),
            pltpu.VMEM((RB, d), jnp.float32),
            pltpu.VMEM_SHARED((v, d), jnp.float32),
            pltpu.SemaphoreType.DMA,
            pltpu.SemaphoreType.DMA,
        ],
    )
    def gather_kernel(idx_hbm, table_hbm, out_hbm, idx_v, rows0, rows1,
                      table_sh, sem0, sem1):
        cid = lax.axis_index("c")
        sid = lax.axis_index("s")
        wid = sid * 2 + cid
        is_extra = wid < extra
        nfull = jnp.where(is_extra, bs + 1, bs)
        base = jnp.where(
            is_extra, wid * stage, off_extra + (wid - extra) * (bs * RB)
        )
        base = pl.multiple_of(base, 8)
        vt = v // 16
        pltpu.sync_copy(
            table_hbm.at[pl.ds(sid * vt, vt)], table_sh.at[pl.ds(sid * vt, vt)]
        )
        pltpu.sync_copy(idx_hbm.at[pl.ds(base, stage)], idx_v)
        plsc.subcore_barrier()
        table_hbm = table_sh

        rows = (rows0, rows1)
        sems = (sem0, sem1)
        pltpu.async_copy(table_hbm.at[idx_v.at[pl.ds(0, RB)]], rows0, sem0)

        def body(s, _):
            def step(c):
                @pl.when(s + 1 < nfull)
                def _():
                    pltpu.async_copy(
                        table_hbm.at[idx_v.at[pl.ds((s + 1) * RB, RB)]],
                        rows[1 - c],
                        sems[1 - c],
                    )
                pltpu.make_async_copy(
                    table_hbm.at[idx_v.at[pl.ds(s * RB, RB)]],
                    rows[c],
                    sems[c],
                ).wait()
                pltpu.sync_copy(
                    rows[c], out_hbm.at[pl.ds(base + s * RB, RB)]
                )

            cur = lax.rem(s, 2)

            @pl.when(cur == 0)
            def _():
                step(0)

            @pl.when(cur == 1)
            def _():
                step(1)

            return _

        lax.fori_loop(0, nfull, body, None)

        if rem:
            @pl.when(wid == NW - 1)
            def _():
                pltpu.async_copy(
                    table_hbm.at[idx_v.at[pl.ds(bs * RB, rem)]],
                    rows[par].at[pl.ds(0, rem)],
                    sems[par],
                )
                pltpu.make_async_copy(
                    table_hbm.at[idx_v.at[pl.ds(bs * RB, rem)]],
                    rows[par].at[pl.ds(0, rem)],
                    sems[par],
                ).wait()
                pltpu.sync_copy(
                    rows[par].at[pl.ds(0, rem)],
                    out_hbm.at[pl.ds(tail_off, rem)],
                )

    return gather_kernel, pad_to


def kernel(x, inf_embed):
    n = x.shape[0]
    v, d = inf_embed.shape
    idx = x.astype(jnp.int32)
    gk, pad_to = _build(n, v, d)
    if pad_to > n:
        idx = jnp.concatenate([idx, jnp.zeros((pad_to - n,), jnp.int32)])
    return gk(idx, inf_embed)


# speedup vs baseline: 5.6315x; 1.0153x over previous
import functools

import jax
import jax.numpy as jnp
from jax import lax
from jax.experimental import pallas as pl
from jax.experimental.pallas import tpu as pltpu
from jax.experimental.pallas import tpu_sc as plsc

NW = 32
RB = 128
NB = 4


@functools.lru_cache(maxsize=None)
def _build(n: int, v: int, d: int):
    assert n % 8 == 0 and n >= NW * RB * NB and v % 16 == 0
    nblk, rem = divmod(n, RB)
    bs, extra = divmod(nblk, NW)
    stage = (bs + 1) * RB
    off_extra = extra * stage
    o_last = off_extra + (NW - 1 - extra) * bs * RB
    tail_off = o_last + bs * RB
    pad_to = o_last + stage

    mesh = plsc.VectorSubcoreMesh(core_axis_name="c", subcore_axis_name="s")

    @functools.partial(
        pl.kernel,
        mesh=mesh,
        out_type=jax.ShapeDtypeStruct((n, d), jnp.float32),
        scratch_types=[
            pltpu.VMEM((stage,), jnp.int32),
            pltpu.VMEM((NB, RB, d), jnp.float32),
            pltpu.VMEM_SHARED((v, d), jnp.float32),
        ]
        + [pltpu.SemaphoreType.DMA] * (2 * NB),
    )
    def gather_kernel(idx_hbm, table_hbm, out_hbm, idx_v, rows_v, table_sh,
                      *sems):
        gsem = sems[:NB]
        wsem = sems[NB:]
        cid = lax.axis_index("c")
        sid = lax.axis_index("s")
        wid = sid * 2 + cid
        is_extra = wid < extra
        nfull = jnp.where(is_extra, bs + 1, bs)
        base = jnp.where(
            is_extra, wid * stage, off_extra + (wid - extra) * (bs * RB)
        )
        base = pl.multiple_of(base, 8)
        vt = v // 16
        pltpu.sync_copy(
            table_hbm.at[pl.ds(sid * vt, vt)], table_sh.at[pl.ds(sid * vt, vt)]
        )
        pltpu.sync_copy(idx_hbm.at[pl.ds(base, stage)], idx_v)
        plsc.subcore_barrier()

        def start_gather(s, b):
            pltpu.async_copy(
                table_sh.at[idx_v.at[pl.ds(s * RB, RB)]], rows_v.at[b], gsem[b]
            )

        def wait_gather(s, b):
            pltpu.make_async_copy(
                table_sh.at[idx_v.at[pl.ds(s * RB, RB)]], rows_v.at[b], gsem[b]
            ).wait()

        def start_write(s, b):
            pltpu.async_copy(
                rows_v.at[b], out_hbm.at[pl.ds(base + s * RB, RB)], wsem[b]
            )

        def wait_write(b):
            pltpu.make_async_copy(
                rows_v.at[b], out_hbm.at[pl.ds(0, RB)], wsem[b]
            ).wait()

        for b in range(NB):
            start_gather(b, b)

        def body(s, _):
            def step(c):
                cg = (c + 2) % NB
                @pl.when((s >= 2) & (s + 2 < nfull))
                def _():
                    wait_write(cg)
                    start_gather(s + 2, cg)

                wait_gather(s, c)
                start_write(s, c)

            cur = lax.rem(s, NB)
            for c in range(NB):
                @pl.when(cur == c)
                def _(c=c):
                    step(c)

            return _

        lax.fori_loop(0, nfull, body, None)

        for b in range(NB):
            wait_write(b)

        if rem:
            @pl.when(wid == NW - 1)
            def _():
                pltpu.async_copy(
                    table_sh.at[idx_v.at[pl.ds(bs * RB, rem)]],
                    rows_v.at[0].at[pl.ds(0, rem)],
                    gsem[0],
                )
                pltpu.make_async_copy(
                    table_sh.at[idx_v.at[pl.ds(bs * RB, rem)]],
                    rows_v.at[0].at[pl.ds(0, rem)],
                    gsem[0],
                ).wait()
                pltpu.sync_copy(
                    rows_v.at[0].at[pl.ds(0, rem)],
                    out_hbm.at[pl.ds(tail_off, rem)],
                )

    return gather_kernel, pad_to


def kernel(x, inf_embed):
    n = x.shape[0]
    v, d = inf_embed.shape
    idx = x.astype(jnp.int32)
    gk, pad_to = _build(n, v, d)
    if pad_to > n:
        idx = jnp.concatenate([idx, jnp.zeros((pad_to - n,), jnp.int32)])
    return gk(idx, inf_embed)


# speedup vs baseline: 5.6466x; 1.0027x over previous
import functools

import jax
import jax.numpy as jnp
from jax import lax
from jax.experimental import pallas as pl
from jax.experimental.pallas import tpu as pltpu
from jax.experimental.pallas import tpu_sc as plsc

NW = 32
RB = 128
NB = 4


@functools.lru_cache(maxsize=None)
def _build(n: int, v: int, d: int):
    assert n % 8 == 0 and n >= NW * RB * NB and v % 16 == 0
    nblk, rem = divmod(n, RB)
    bs, extra = divmod(nblk, NW)
    stage = (bs + 1) * RB
    off_extra = extra * stage
    o_last = off_extra + (NW - 1 - extra) * bs * RB
    tail_off = o_last + bs * RB
    last_len = bs * RB + rem
    assert last_len % 8 == 0

    mesh = plsc.VectorSubcoreMesh(core_axis_name="c", subcore_axis_name="s")

    @functools.partial(
        pl.kernel,
        mesh=mesh,
        out_type=jax.ShapeDtypeStruct((n, d), jnp.float32),
        scratch_types=[
            pltpu.VMEM((stage,), jnp.int32),
            pltpu.VMEM((NB, RB, d), jnp.float32),
            pltpu.VMEM_SHARED((v, d), jnp.float32),
        ]
        + [pltpu.SemaphoreType.DMA] * (2 * NB),
    )
    def gather_kernel(idx_hbm, table_hbm, out_hbm, idx_v, rows_v, table_sh,
                      *sems):
        gsem = sems[:NB]
        wsem = sems[NB:]
        cid = lax.axis_index("c")
        sid = lax.axis_index("s")
        wid = sid * 2 + cid
        is_extra = wid < extra
        nfull = jnp.where(is_extra, bs + 1, bs)
        base = jnp.where(
            is_extra, wid * stage, off_extra + (wid - extra) * (bs * RB)
        )
        base = pl.multiple_of(base, 8)
        vt = v // 16
        pltpu.sync_copy(
            table_hbm.at[pl.ds(sid * vt, vt)], table_sh.at[pl.ds(sid * vt, vt)]
        )
        @pl.when(wid < NW - 1)
        def _():
            pltpu.sync_copy(idx_hbm.at[pl.ds(base, stage)], idx_v)

        @pl.when(wid == NW - 1)
        def _():
            pltpu.sync_copy(
                idx_hbm.at[pl.ds(o_last, last_len)],
                idx_v.at[pl.ds(0, last_len)],
            )

        plsc.subcore_barrier()

        def start_gather(s, b):
            pltpu.async_copy(
                table_sh.at[idx_v.at[pl.ds(s * RB, RB)]], rows_v.at[b], gsem[b]
            )

        def wait_gather(s, b):
            pltpu.make_async_copy(
                table_sh.at[idx_v.at[pl.ds(s * RB, RB)]], rows_v.at[b], gsem[b]
            ).wait()

        def start_write(s, b):
            pltpu.async_copy(
                rows_v.at[b], out_hbm.at[pl.ds(base + s * RB, RB)], wsem[b]
            )

        def wait_write(b):
            pltpu.make_async_copy(
                rows_v.at[b], out_hbm.at[pl.ds(0, RB)], wsem[b]
            ).wait()

        for b in range(NB):
            start_gather(b, b)

        def body(s, _):
            def step(c):
                cg = (c + 2) % NB
                @pl.when((s >= 2) & (s + 2 < nfull))
                def _():
                    wait_write(cg)
                    start_gather(s + 2, cg)

                wait_gather(s, c)
                start_write(s, c)

            cur = lax.rem(s, NB)
            for c in range(NB):
                @pl.when(cur == c)
                def _(c=c):
                    step(c)

            return _

        lax.fori_loop(0, nfull, body, None)

        for b in range(NB):
            wait_write(b)

        if rem:
            @pl.when(wid == NW - 1)
            def _():
                pltpu.async_copy(
                    table_sh.at[idx_v.at[pl.ds(bs * RB, rem)]],
                    rows_v.at[0].at[pl.ds(0, rem)],
                    gsem[0],
                )
                pltpu.make_async_copy(
                    table_sh.at[idx_v.at[pl.ds(bs * RB, rem)]],
                    rows_v.at[0].at[pl.ds(0, rem)],
                    gsem[0],
                ).wait()
                pltpu.sync_copy(
                    rows_v.at[0].at[pl.ds(0, rem)],
                    out_hbm.at[pl.ds(tail_off, rem)],
                )

    return gather_kernel


def kernel(x, inf_embed):
    n = x.shape[0]
    v, d = inf_embed.shape
    idx = x.astype(jnp.int32)
    return _build(n, v, d)(idx, inf_embed)


# speedup vs baseline: 5.7461x; 1.0176x over previous
import functools

import jax
import jax.numpy as jnp
from jax import lax
from jax.experimental import pallas as pl
from jax.experimental.pallas import tpu as pltpu
from jax.experimental.pallas import tpu_sc as plsc

NW = 32
RB = 128
NB = 4


@functools.lru_cache(maxsize=None)
def _build(n: int, v: int, d: int):
    assert n % 8 == 0 and n >= NW * RB * NB and v % 16 == 0
    nblk, rem = divmod(n, RB)
    bs, extra = divmod(nblk, NW)
    stage = (bs + 1) * RB
    off_extra = extra * stage
    o_last = off_extra + (NW - 1 - extra) * bs * RB
    tail_off = o_last + bs * RB
    last_len = bs * RB + rem
    assert last_len % 8 == 0

    mesh = plsc.VectorSubcoreMesh(core_axis_name="c", subcore_axis_name="s")

    @functools.partial(
        pl.kernel,
        mesh=mesh,
        out_type=jax.ShapeDtypeStruct((n, d), jnp.float32),
        scratch_types=[
            pltpu.VMEM((stage,), jnp.int32),
            pltpu.VMEM((NB, RB, d), jnp.float32),
            pltpu.VMEM_SHARED((v, d), jnp.float32),
        ]
        + [pltpu.SemaphoreType.DMA] * (2 * NB),
    )
    def gather_kernel(idx_hbm, table_hbm, out_hbm, idx_v, rows_v, table_sh,
                      *sems):
        gsem = sems[:NB]
        wsem = sems[NB:]
        cid = lax.axis_index("c")
        sid = lax.axis_index("s")
        wid = sid * 2 + cid
        is_extra = wid < extra
        nfull = jnp.where(is_extra, bs + 1, bs)
        base = jnp.where(
            is_extra, wid * stage, off_extra + (wid - extra) * (bs * RB)
        )
        base = pl.multiple_of(base, 8)
        vt = v // 16
        sems_ = sems
        pltpu.async_copy(
            table_hbm.at[pl.ds(sid * vt, vt)],
            table_sh.at[pl.ds(sid * vt, vt)],
            sems_[0],
        )
        @pl.when(wid < NW - 1)
        def _():
            pltpu.async_copy(idx_hbm.at[pl.ds(base, stage)], idx_v, sems_[1])
            pltpu.make_async_copy(
                idx_hbm.at[pl.ds(base, stage)], idx_v, sems_[1]
            ).wait()

        @pl.when(wid == NW - 1)
        def _():
            pltpu.async_copy(
                idx_hbm.at[pl.ds(o_last, last_len)],
                idx_v.at[pl.ds(0, last_len)],
                sems_[1],
            )
            pltpu.make_async_copy(
                idx_hbm.at[pl.ds(o_last, last_len)],
                idx_v.at[pl.ds(0, last_len)],
                sems_[1],
            ).wait()

        pltpu.make_async_copy(
            table_hbm.at[pl.ds(sid * vt, vt)],
            table_sh.at[pl.ds(sid * vt, vt)],
            sems_[0],
        ).wait()
        plsc.subcore_barrier()

        def start_gather(s, b):
            pltpu.async_copy(
                table_sh.at[idx_v.at[pl.ds(s * RB, RB)]], rows_v.at[b], gsem[b]
            )

        def wait_gather(s, b):
            pltpu.make_async_copy(
                table_sh.at[idx_v.at[pl.ds(s * RB, RB)]], rows_v.at[b], gsem[b]
            ).wait()

        def start_write(s, b):
            pltpu.async_copy(
                rows_v.at[b], out_hbm.at[pl.ds(base + s * RB, RB)], wsem[b]
            )

        def wait_write(b):
            pltpu.make_async_copy(
                rows_v.at[b], out_hbm.at[pl.ds(0, RB)], wsem[b]
            ).wait()

        for b in range(NB):
            start_gather(b, b)

        def body(s, _):
            def step(c):
                cg = (c + 2) % NB
                @pl.when((s >= 2) & (s + 2 < nfull))
                def _():
                    wait_write(cg)
                    start_gather(s + 2, cg)

                wait_gather(s, c)
                start_write(s, c)

            cur = lax.rem(s, NB)
            for c in range(NB):
                @pl.when(cur == c)
                def _(c=c):
                    step(c)

            return _

        lax.fori_loop(0, nfull, body, None)

        for b in range(NB):
            wait_write(b)

        if rem:
            @pl.when(wid == NW - 1)
            def _():
                pltpu.async_copy(
                    table_sh.at[idx_v.at[pl.ds(bs * RB, rem)]],
                    rows_v.at[0].at[pl.ds(0, rem)],
                    gsem[0],
                )
                pltpu.make_async_copy(
                    table_sh.at[idx_v.at[pl.ds(bs * RB, rem)]],
                    rows_v.at[0].at[pl.ds(0, rem)],
                    gsem[0],
                ).wait()
                pltpu.sync_copy(
                    rows_v.at[0].at[pl.ds(0, rem)],
                    out_hbm.at[pl.ds(tail_off, rem)],
                )

    return gather_kernel


def kernel(x, inf_embed):
    n = x.shape[0]
    v, d = inf_embed.shape
    idx = x.astype(jnp.int32)
    return _build(n, v, d)(idx, inf_embed)


# speedup vs baseline: 5.7634x; 1.0030x over previous
import functools

import jax
import jax.numpy as jnp
from jax import lax
from jax.experimental import pallas as pl
from jax.experimental.pallas import tpu as pltpu
from jax.experimental.pallas import tpu_sc as plsc

NW = 32
RB = 128
NB = 4


@functools.lru_cache(maxsize=None)
def _build(n: int, v: int, d: int):
    assert n % 8 == 0 and v % 16 == 0
    q8, r8 = divmod(n // 8, NW)
    cb = q8 * 8
    ca = cb + 8
    fa, pa = divmod(ca, RB)
    fb, pb = divmod(cb, RB)
    assert fa >= NB and fb >= NB
    off_b = r8 * ca

    mesh = plsc.VectorSubcoreMesh(core_axis_name="c", subcore_axis_name="s")

    @functools.partial(
        pl.kernel,
        mesh=mesh,
        out_type=jax.ShapeDtypeStruct((n, d), jnp.float32),
        scratch_types=[
            pltpu.VMEM((ca,), jnp.int32),
            pltpu.VMEM((NB, RB, d), jnp.float32),
            pltpu.VMEM_SHARED((v, d), jnp.float32),
        ]
        + [pltpu.SemaphoreType.DMA] * (2 * NB),
    )
    def gather_kernel(idx_hbm, table_hbm, out_hbm, idx_v, rows_v, table_sh,
                      *sems):
        gsem = sems[:NB]
        wsem = sems[NB:]
        cid = lax.axis_index("c")
        sid = lax.axis_index("s")
        wid = sid * 2 + cid
        is_a = wid < r8
        nfull = jnp.where(is_a, fa, fb)
        base = jnp.where(is_a, wid * ca, off_b + (wid - r8) * cb)
        base = pl.multiple_of(base, 8)

        vt = v // 16
        pltpu.async_copy(
            table_hbm.at[pl.ds(sid * vt, vt)],
            table_sh.at[pl.ds(sid * vt, vt)],
            gsem[0],
        )

        def stage_idx(count):
            pltpu.async_copy(
                idx_hbm.at[pl.ds(base, count)],
                idx_v.at[pl.ds(0, count)],
                gsem[1],
            )
            pltpu.make_async_copy(
                idx_hbm.at[pl.ds(base, count)],
                idx_v.at[pl.ds(0, count)],
                gsem[1],
            ).wait()

        @pl.when(is_a)
        def _():
            stage_idx(ca)

        @pl.when(jnp.logical_not(is_a))
        def _():
            stage_idx(cb)

        pltpu.make_async_copy(
            table_hbm.at[pl.ds(sid * vt, vt)],
            table_sh.at[pl.ds(sid * vt, vt)],
            gsem[0],
        ).wait()
        plsc.subcore_barrier()

        def start_gather(s, b):
            pltpu.async_copy(
                table_sh.at[idx_v.at[pl.ds(s * RB, RB)]], rows_v.at[b], gsem[b]
            )

        def wait_gather(s, b):
            pltpu.make_async_copy(
                table_sh.at[idx_v.at[pl.ds(s * RB, RB)]], rows_v.at[b], gsem[b]
            ).wait()

        def start_write(s, b):
            pltpu.async_copy(
                rows_v.at[b], out_hbm.at[pl.ds(base + s * RB, RB)], wsem[b]
            )

        def wait_write(b):
            pltpu.make_async_copy(
                rows_v.at[b], out_hbm.at[pl.ds(0, RB)], wsem[b]
            ).wait()

        for b in range(NB):
            start_gather(b, b)

        def body(s, _):
            def step(c):
                cg = (c + 2) % NB
                @pl.when((s >= 2) & (s + 2 < nfull))
                def _():
                    wait_write(cg)
                    start_gather(s + 2, cg)

                wait_gather(s, c)
                start_write(s, c)

            cur = lax.rem(s, NB)
            for c in range(NB):
                @pl.when(cur == c)
                def _(c=c):
                    step(c)

            return _

        lax.fori_loop(0, nfull, body, None)

        def epilogue(f, p):
            c_p = f % NB
            if p:
                wait_write(c_p)
                pltpu.async_copy(
                    table_sh.at[idx_v.at[pl.ds(f * RB, p)]],
                    rows_v.at[c_p].at[pl.ds(0, p)],
                    gsem[c_p],
                )
                pltpu.make_async_copy(
                    table_sh.at[idx_v.at[pl.ds(f * RB, p)]],
                    rows_v.at[c_p].at[pl.ds(0, p)],
                    gsem[c_p],
                ).wait()
                pltpu.async_copy(
                    rows_v.at[c_p].at[pl.ds(0, p)],
                    out_hbm.at[pl.ds(base + f * RB, p)],
                    wsem[c_p],
                )
            for b in range(NB):
                if p and b == c_p:
                    continue
                wait_write(b)
            if p:
                pltpu.make_async_copy(
                    rows_v.at[c_p].at[pl.ds(0, p)],
                    out_hbm.at[pl.ds(0, p)],
                    wsem[c_p],
                ).wait()

        @pl.when(is_a)
        def _():
            epilogue(fa, pa)

        @pl.when(jnp.logical_not(is_a))
        def _():
            epilogue(fb, pb)

    return gather_kernel


def kernel(x, inf_embed):
    n = x.shape[0]
    v, d = inf_embed.shape
    idx = x.astype(jnp.int32)
    return _build(n, v, d)(idx, inf_embed)
